# baseline (speedup 1.0000x reference)
import functools
import math

import jax
import jax.numpy as jnp
from jax import lax
from jax.experimental import pallas as pl
from jax.experimental.pallas import tpu as pltpu
from jax.experimental.pallas import tpu_sc as plsc

EMBED_DIM = 64
EMBED_SCALE = math.sqrt(EMBED_DIM)
CHUNK = 256
NBUF = 2


@jax.jit
def _sc_scaled_gather(tokens_t, weight):
    S, N = tokens_t.shape
    info = plsc.get_sparse_core_info()
    nw = info.num_cores * info.num_subcores
    blocks_per_s = N // CHUNK
    n_chunks_total = S * blocks_per_s
    chunks_per_w = n_chunks_total // nw
    n_super = chunks_per_w // NBUF
    assert blocks_per_s * CHUNK == N
    assert chunks_per_w * nw == n_chunks_total
    assert n_super * NBUF == chunks_per_w
    assert blocks_per_s & (blocks_per_s - 1) == 0
    blk_bits = blocks_per_s.bit_length() - 1

    mesh = plsc.VectorSubcoreMesh(core_axis_name="c", subcore_axis_name="s")

    @functools.partial(
        pl.kernel,
        mesh=mesh,
        out_type=jax.ShapeDtypeStruct((N, S, EMBED_DIM), jnp.float32),
        scratch_types=[
            *[pltpu.VMEM((CHUNK,), jnp.int32) for _ in range(NBUF)],
            *[pltpu.VMEM((CHUNK, EMBED_DIM), jnp.float32) for _ in range(NBUF)],
            *[pltpu.VMEM((CHUNK, EMBED_DIM), jnp.float32) for _ in range(NBUF)],
            *[pltpu.SemaphoreType.DMA for _ in range(2 * NBUF)],
        ],
        compiler_params=pltpu.CompilerParams(use_tc_tiling_on_sc=False),
    )
    def k(tok_hbm, table_hbm, out_hbm, i0, i1, in0, in1, o0, o1, g0, g1, s0, s1):
        idx_v = (i0, i1)
        in_v = (in0, in1)
        out_v = (o0, o1)
        gsem = (g0, g1)
        ssem = (s0, s1)
        wid = lax.axis_index("s") * info.num_cores + lax.axis_index("c")
        base = wid * chunks_per_w

        def fire_gather(b, chunk_id):
            s_pos = chunk_id >> blk_bits
            n0 = (chunk_id & (blocks_per_s - 1)) * CHUNK
            pltpu.sync_copy(tok_hbm.at[s_pos, pl.ds(n0, CHUNK)], idx_v[b])
            pltpu.async_copy(table_hbm.at[idx_v[b]], in_v[b], gsem[b])

        for b in range(NBUF):
            fire_gather(b, base + b)

        def super_body(g, carry):
            for b in range(NBUF):
                cid = base + g * NBUF + b
                pltpu.make_async_copy(
                    table_hbm.at[idx_v[b]], in_v[b], gsem[b]
                ).wait()

                @pl.when(g > 0)
                def _():
                    pltpu.make_async_copy(
                        out_v[b], out_hbm.at[pl.ds(0, CHUNK), 0], ssem[b]
                    ).wait()

                @plsc.parallel_loop(0, CHUNK, unroll=4)
                def _(r):
                    for j in range(EMBED_DIM // 16):
                        sl = pl.ds(j * 16, 16)
                        out_v[b][r, sl] = in_v[b][r, sl] * EMBED_SCALE

                s_pos = cid >> blk_bits
                n0 = (cid & (blocks_per_s - 1)) * CHUNK
                pltpu.async_copy(
                    out_v[b], out_hbm.at[pl.ds(n0, CHUNK), s_pos], ssem[b]
                )

                @pl.when(g + 1 < n_super)
                def _():
                    fire_gather(b, cid + NBUF)
            return carry

        lax.fori_loop(0, n_super, super_body, 0)

        for b in range(NBUF):
            pltpu.make_async_copy(
                out_v[b], out_hbm.at[pl.ds(0, CHUNK), 0], ssem[b]
            ).wait()

    return k(tokens_t, weight)


def kernel(tokens, weight):
    out = _sc_scaled_gather(tokens.T, weight)
    return out
